# baseline (speedup 1.0000x reference)
import jax
import jax.numpy as jnp
from jax.experimental import pallas as pl
from jax.experimental.pallas import tpu as pltpu


def _mm_kernel(x_ref, w_ref, o_ref):
    o_ref[...] = jnp.dot(x_ref[...], w_ref[...],
                         preferred_element_type=jnp.float32)


def _layer1_kernel(adj_ref, s_ref, b_ref, o_ref):
    t = jnp.dot(adj_ref[...], s_ref[...],
                preferred_element_type=jnp.float32)
    o_ref[...] = jnp.maximum(t + b_ref[...], 0.0)


def _layer2_kernel(adj_ref, h_ref, w2_ref, b2_ref, o_ref):
    g = jnp.dot(adj_ref[...], h_ref[...],
                preferred_element_type=jnp.float32)
    z = jnp.dot(g, w2_ref[...],
                preferred_element_type=jnp.float32) + b2_ref[...]
    z = jnp.maximum(z, 0.0)
    o_ref[0, :, :] = jnp.sum(z, axis=0, keepdims=True)


def kernel(x, adj, W1, b1, W2, b2):
    batch, n, nfeat = x.shape
    nhid = W1.shape[1]
    x2 = x.reshape(n, nfeat)
    adj2 = adj.reshape(n, n)

    bi = 400
    ni = n // bi

    s1 = pl.pallas_call(
        _mm_kernel,
        grid=(ni,),
        in_specs=[
            pl.BlockSpec((bi, nfeat), lambda i: (i, 0)),
            pl.BlockSpec((nfeat, nhid), lambda i: (0, 0)),
        ],
        out_specs=pl.BlockSpec((bi, nhid), lambda i: (i, 0)),
        out_shape=jax.ShapeDtypeStruct((n, nhid), jnp.float32),
    )(x2, W1)

    h = pl.pallas_call(
        _layer1_kernel,
        grid=(ni,),
        in_specs=[
            pl.BlockSpec((bi, n), lambda i: (i, 0)),
            pl.BlockSpec((n, nhid), lambda i: (0, 0)),
            pl.BlockSpec((1, nhid), lambda i: (0, 0)),
        ],
        out_specs=pl.BlockSpec((bi, nhid), lambda i: (i, 0)),
        out_shape=jax.ShapeDtypeStruct((n, nhid), jnp.float32),
        compiler_params=pltpu.CompilerParams(
            dimension_semantics=("arbitrary",)),
    )(adj2, s1, b1.reshape(1, nhid))

    partials = pl.pallas_call(
        _layer2_kernel,
        grid=(ni,),
        in_specs=[
            pl.BlockSpec((bi, n), lambda i: (i, 0)),
            pl.BlockSpec((n, nhid), lambda i: (0, 0)),
            pl.BlockSpec((nhid, nfeat), lambda i: (0, 0)),
            pl.BlockSpec((1, nfeat), lambda i: (0, 0)),
        ],
        out_specs=pl.BlockSpec((1, 1, nfeat), lambda i: (i, 0, 0)),
        out_shape=jax.ShapeDtypeStruct((ni, 1, nfeat), jnp.float32),
        compiler_params=pltpu.CompilerParams(
            dimension_semantics=("arbitrary",)),
    )(adj2, h, W2, b2.reshape(1, nfeat))

    return (jnp.sum(partials) / (n * nfeat)).reshape(batch)


# speedup vs baseline: 1.0099x; 1.0099x over previous
import jax
import jax.numpy as jnp
from jax.experimental import pallas as pl
from jax.experimental.pallas import tpu as pltpu


def _mm_kernel(x_ref, w_ref, o_ref):
    o_ref[...] = jnp.dot(x_ref[...], w_ref[...],
                         preferred_element_type=jnp.float32
                         ).astype(jnp.bfloat16)


def _layer1_kernel(adj_ref, s_ref, b_ref, o_ref):
    t = jnp.dot(adj_ref[...].astype(jnp.bfloat16), s_ref[...],
                preferred_element_type=jnp.float32)
    o_ref[...] = jnp.maximum(t + b_ref[...], 0.0).astype(jnp.bfloat16)


def _layer2_kernel(adj_ref, h_ref, w2_ref, b2_ref, o_ref):
    g = jnp.dot(adj_ref[...].astype(jnp.bfloat16), h_ref[...],
                preferred_element_type=jnp.float32)
    z = jnp.dot(g, w2_ref[...],
                preferred_element_type=jnp.float32) + b2_ref[...]
    z = jnp.maximum(z, 0.0)
    o_ref[0, :, :] = jnp.sum(z, axis=0, keepdims=True)


def kernel(x, adj, W1, b1, W2, b2):
    batch, n, nfeat = x.shape
    nhid = W1.shape[1]
    x2 = x.reshape(n, nfeat)
    adj2 = adj.reshape(n, n)

    bi = 400
    ni = n // bi

    s1 = pl.pallas_call(
        _mm_kernel,
        grid=(ni,),
        in_specs=[
            pl.BlockSpec((bi, nfeat), lambda i: (i, 0)),
            pl.BlockSpec((nfeat, nhid), lambda i: (0, 0)),
        ],
        out_specs=pl.BlockSpec((bi, nhid), lambda i: (i, 0)),
        out_shape=jax.ShapeDtypeStruct((n, nhid), jnp.bfloat16),
    )(x2, W1)

    h = pl.pallas_call(
        _layer1_kernel,
        grid=(ni,),
        in_specs=[
            pl.BlockSpec((bi, n), lambda i: (i, 0)),
            pl.BlockSpec((n, nhid), lambda i: (0, 0)),
            pl.BlockSpec((1, nhid), lambda i: (0, 0)),
        ],
        out_specs=pl.BlockSpec((bi, nhid), lambda i: (i, 0)),
        out_shape=jax.ShapeDtypeStruct((n, nhid), jnp.bfloat16),
        compiler_params=pltpu.CompilerParams(
            dimension_semantics=("arbitrary",)),
    )(adj2, s1, b1.reshape(1, nhid))

    partials = pl.pallas_call(
        _layer2_kernel,
        grid=(ni,),
        in_specs=[
            pl.BlockSpec((bi, n), lambda i: (i, 0)),
            pl.BlockSpec((n, nhid), lambda i: (0, 0)),
            pl.BlockSpec((nhid, nfeat), lambda i: (0, 0)),
            pl.BlockSpec((1, nfeat), lambda i: (0, 0)),
        ],
        out_specs=pl.BlockSpec((1, 1, nfeat), lambda i: (i, 0, 0)),
        out_shape=jax.ShapeDtypeStruct((ni, 1, nfeat), jnp.float32),
        compiler_params=pltpu.CompilerParams(
            dimension_semantics=("arbitrary",)),
    )(adj2, h, W2, b2.reshape(1, nfeat))

    return (jnp.sum(partials) / (n * nfeat)).reshape(batch)
